# baseline (speedup 1.0000x reference)
import functools

import jax
import jax.numpy as jnp
from jax import lax
from jax.experimental import pallas as pl
from jax.experimental.pallas import tpu as pltpu
from jax.experimental.pallas import tpu_sc as plsc

HIDDEN = 128
N_NODE_FEAT = 128
N_EDGE_FEAT = 16
N_NODES = 10000
N_EDGES = 320000


_ROWS_BLK = 1000


def _p_table_body(x_ref, wc_ref, bc_ref, t_ref):
    t_ref[...] = (
        jnp.dot(x_ref[...], wc_ref[...], preferred_element_type=jnp.float32,
                precision=lax.Precision.HIGHEST)
        + bc_ref[...]
    )


def _p_table(emb, wc, bc):
    grid = (N_NODES // _ROWS_BLK,)
    return pl.pallas_call(
        _p_table_body,
        grid=grid,
        in_specs=[
            pl.BlockSpec((_ROWS_BLK, HIDDEN), lambda i: (i, 0)),
            pl.BlockSpec((HIDDEN, HIDDEN), lambda i: (0, 0)),
            pl.BlockSpec((1, HIDDEN), lambda i: (0, 0)),
        ],
        out_specs=pl.BlockSpec((_ROWS_BLK, HIDDEN), lambda i: (i, 0)),
        out_shape=jax.ShapeDtypeStruct((N_NODES, HIDDEN), jnp.float32),
    )(emb, wc, bc)


def _node_body(x_ref, w_ref, b_ref, o_ref):
    o_ref[...] = (
        jnp.dot(x_ref[...], w_ref[...], preferred_element_type=jnp.float32,
                precision=lax.Precision.HIGHEST)
        + b_ref[...]
    )


def _node_linear(emb, w, b):
    grid = (N_NODES // _ROWS_BLK,)
    return pl.pallas_call(
        _node_body,
        grid=grid,
        in_specs=[
            pl.BlockSpec((_ROWS_BLK, HIDDEN), lambda i: (i, 0)),
            pl.BlockSpec((HIDDEN, N_NODE_FEAT), lambda i: (0, 0)),
            pl.BlockSpec((1, N_NODE_FEAT), lambda i: (0, 0)),
        ],
        out_specs=pl.BlockSpec((_ROWS_BLK, N_NODE_FEAT), lambda i: (i, 0)),
        out_shape=jax.ShapeDtypeStruct((N_NODES, N_NODE_FEAT), jnp.float32),
    )(emb, w, b)


_NC = 2
_NS = 16
_NW = _NC * _NS
_BLK = 128
_NBLOCKS = N_EDGES // _BLK
_BPW = _NBLOCKS // _NW
_XTRA = _NBLOCKS - _BPW * _NW
_TILE_W = 8 * _BLK
_OUT_WORDS = N_EDGES * N_EDGE_FEAT
_HALF = _NBLOCKS * _TILE_W


def _edge_body(t_hbm, idx_hbm, out_hbm,
               slab_v, xtra_v,
               a0, a1, b0, b1, o0, o1, s0, s1,
               gs0, gs1, os0, os1):
    wid = lax.axis_index("s") * _NC + lax.axis_index("c")
    blk0 = wid * _BPW

    pltpu.sync_copy(idx_hbm.at[pl.ds(blk0, _BPW)], slab_v)

    cvec = lax.iota(jnp.int32, 16)

    @pl.loop(0, _BPW)
    def _(j):
        for r in range(2):
            for v in range(8):
                sl = pl.ds(v * 16, 16)
                slab_v[j, r, sl] = slab_v[j, r, sl] * 8 + r

    abufs = (a0, a1)
    bbufs = (b0, b1)
    obufs = (o0, o1)
    sbufs = (s0, s1)
    gsems = (gs0, gs1)
    osems = (os0, os1)

    def issue_gather(j, k):
        pltpu.async_copy(t_hbm.at[slab_v.at[j, 0]], abufs[k], gsems[k])
        pltpu.async_copy(t_hbm.at[slab_v.at[j, 1]], bbufs[k], gsems[k])

    def wait_gather(k):
        pltpu.make_async_copy(t_hbm.at[pl.ds(0, _BLK)], abufs[k],
                              gsems[k]).wait()
        pltpu.make_async_copy(t_hbm.at[pl.ds(0, _BLK)], bbufs[k],
                              gsems[k]).wait()

    iota17 = cvec * 17

    def compute(k):
        a_ref, b_ref, o_ref = abufs[k], bbufs[k], obufs[k]

        @pl.loop(0, 8)
        def _(g):
            rowv = cvec + g * 16
            gout = g * 16
            for c in range(16):
                colv = jnp.full((16,), c, jnp.int32)
                va = plsc.load_gather(a_ref, [rowv, colv])
                vb = plsc.load_gather(b_ref, [rowv, colv])
                dst = (c >> 3) * _TILE_W + (c & 7) * _BLK
                o_ref[pl.ds(gout + dst, 16)] = va + vb

    def issue_out(j, k):
        b = blk0 + j
        o_ref = obufs[k]
        pltpu.async_copy(o_ref.at[pl.ds(0, _TILE_W)],
                         out_hbm.at[pl.ds(b * _TILE_W, _TILE_W)], osems[k])
        pltpu.async_copy(o_ref.at[pl.ds(_TILE_W, _TILE_W)],
                         out_hbm.at[pl.ds(_HALF + b * _TILE_W, _TILE_W)],
                         osems[k])

    def wait_out(k):
        pltpu.make_async_copy(obufs[k], out_hbm.at[pl.ds(0, 2 * _TILE_W)],
                              osems[k]).wait()

    issue_gather(0, 0)
    issue_gather(1, 1)

    for j in (0, 1):
        k = j % 2
        wait_gather(k)
        compute(k)
        issue_out(j, k)
        issue_gather(j + 2, k)

    @pl.loop(2, _BPW - 2, step=2)
    def _(c):
        for k in range(2):
            j = c + k
            wait_gather(k)
            wait_out(k)
            compute(k)
            issue_out(j, k)
            issue_gather(j + 2, k)

    for j in (_BPW - 2, _BPW - 1):
        k = j % 2
        wait_gather(k)
        wait_out(k)
        compute(k)
        issue_out(j, k)

    wait_out(0)
    wait_out(1)

    @pl.when(wid < _XTRA)
    def _():
        xb = _NW * _BPW + wid
        pltpu.sync_copy(idx_hbm.at[xb], xtra_v)
        for r in range(2):
            for v in range(8):
                sl = pl.ds(v * 16, 16)
                xtra_v[r, sl] = xtra_v[r, sl] * 8 + r
        pltpu.async_copy(t_hbm.at[xtra_v.at[0]], a0, gs0)
        pltpu.async_copy(t_hbm.at[xtra_v.at[1]], b0, gs0)
        wait_gather(0)
        compute(0)
        pltpu.async_copy(o0.at[pl.ds(0, _TILE_W)],
                         out_hbm.at[pl.ds(xb * _TILE_W, _TILE_W)], os0)
        pltpu.async_copy(o0.at[pl.ds(_TILE_W, _TILE_W)],
                         out_hbm.at[pl.ds(_HALF + xb * _TILE_W, _TILE_W)],
                         os0)
        wait_out(0)


def _edge_decode(t2, idx3):
    mesh = plsc.VectorSubcoreMesh(core_axis_name="c", subcore_axis_name="s")
    f32 = jnp.float32
    run = pl.kernel(
        _edge_body,
        out_type=jax.ShapeDtypeStruct((_OUT_WORDS,), f32),
        mesh=mesh,
        compiler_params=pltpu.CompilerParams(use_tc_tiling_on_sc=False,
                                             needs_layout_passes=False),
        scratch_types=[
            pltpu.VMEM((_BPW, 2, _BLK), jnp.int32),
            pltpu.VMEM((2, _BLK), jnp.int32),
            pltpu.VMEM((_BLK, N_EDGE_FEAT), f32),
            pltpu.VMEM((_BLK, N_EDGE_FEAT), f32),
            pltpu.VMEM((_BLK, N_EDGE_FEAT), f32),
            pltpu.VMEM((_BLK, N_EDGE_FEAT), f32),
            pltpu.VMEM((2 * _TILE_W,), f32),
            pltpu.VMEM((2 * _TILE_W,), f32),
            pltpu.VMEM((_BLK * 17,), f32),
            pltpu.VMEM((_BLK * 17,), f32),
            pltpu.SemaphoreType.DMA,
            pltpu.SemaphoreType.DMA,
            pltpu.SemaphoreType.DMA,
            pltpu.SemaphoreType.DMA,
        ],
    )
    return run(t2, idx3)


def kernel(embeddings, edge_index, node_w, node_b, edge_w, edge_b):
    pad = HIDDEN - 2 * N_EDGE_FEAT
    wc = jnp.concatenate(
        [edge_w[:, :HIDDEN].T, edge_w[:, HIDDEN:].T,
         jnp.zeros((HIDDEN, pad), jnp.float32)], axis=1)
    bc = jnp.pad(edge_b, (0, HIDDEN - N_EDGE_FEAT)).reshape(1, HIDDEN)

    t = _p_table(embeddings, wc, bc)
    t2 = t.reshape(8 * N_NODES, N_EDGE_FEAT)

    idx3 = (edge_index.astype(jnp.int32)
            .reshape(2, _NBLOCKS, _BLK)
            .transpose(1, 0, 2))

    out_flat = _edge_decode(t2, idx3)
    edge_hat = (out_flat
                .reshape(2, _NBLOCKS, 8, _BLK)
                .transpose(1, 3, 0, 2)
                .reshape(N_EDGES, N_EDGE_FEAT))

    node_hat = _node_linear(embeddings, node_w.T,
                            node_b.reshape(1, N_NODE_FEAT))
    return (node_hat, edge_hat)


# speedup vs baseline: 2.0298x; 2.0298x over previous
import functools

import jax
import jax.numpy as jnp
from jax import lax
from jax.experimental import pallas as pl
from jax.experimental.pallas import tpu as pltpu
from jax.experimental.pallas import tpu_sc as plsc

HIDDEN = 128
N_NODE_FEAT = 128
N_EDGE_FEAT = 16
N_NODES = 10000
N_EDGES = 320000


_ROWS_BLK = 1000


def _p_table_body(x_ref, wc_ref, bc_ref, t_ref):
    t_ref[...] = (
        jnp.dot(x_ref[...], wc_ref[...], preferred_element_type=jnp.float32,
                precision=lax.Precision.HIGHEST)
        + bc_ref[...]
    )


def _p_table(emb, wc, bc):
    grid = (N_NODES // _ROWS_BLK,)
    return pl.pallas_call(
        _p_table_body,
        grid=grid,
        in_specs=[
            pl.BlockSpec((_ROWS_BLK, HIDDEN), lambda i: (i, 0)),
            pl.BlockSpec((HIDDEN, HIDDEN), lambda i: (0, 0)),
            pl.BlockSpec((1, HIDDEN), lambda i: (0, 0)),
        ],
        out_specs=pl.BlockSpec((_ROWS_BLK, HIDDEN), lambda i: (i, 0)),
        out_shape=jax.ShapeDtypeStruct((N_NODES, HIDDEN), jnp.float32),
    )(emb, wc, bc)


def _node_body(x_ref, w_ref, b_ref, o_ref):
    o_ref[...] = (
        jnp.dot(x_ref[...], w_ref[...], preferred_element_type=jnp.float32,
                precision=lax.Precision.HIGHEST)
        + b_ref[...]
    )


def _node_linear(emb, w, b):
    grid = (N_NODES // _ROWS_BLK,)
    return pl.pallas_call(
        _node_body,
        grid=grid,
        in_specs=[
            pl.BlockSpec((_ROWS_BLK, HIDDEN), lambda i: (i, 0)),
            pl.BlockSpec((HIDDEN, N_NODE_FEAT), lambda i: (0, 0)),
            pl.BlockSpec((1, N_NODE_FEAT), lambda i: (0, 0)),
        ],
        out_specs=pl.BlockSpec((_ROWS_BLK, N_NODE_FEAT), lambda i: (i, 0)),
        out_shape=jax.ShapeDtypeStruct((N_NODES, N_NODE_FEAT), jnp.float32),
    )(emb, w, b)


_NC = 2
_NS = 16
_NW = _NC * _NS
_BLK = 128
_NBLOCKS = N_EDGES // _BLK
_BPW = _NBLOCKS // _NW
_XTRA = _NBLOCKS - _BPW * _NW
_TILE_W = 8 * _BLK
_OUT_WORDS = N_EDGES * N_EDGE_FEAT
_HALF = _NBLOCKS * _TILE_W


def _edge_body(t_hbm, idx_hbm, out_hbm,
               slab_v, xtra_v,
               a0, a1, b0, b1, o0, o1, s0, s1,
               gs0, gs1, os0, os1):
    wid = lax.axis_index("s") * _NC + lax.axis_index("c")
    blk0 = wid * _BPW

    pltpu.sync_copy(idx_hbm.at[pl.ds(blk0, _BPW)], slab_v)

    cvec = lax.iota(jnp.int32, 16)

    @pl.loop(0, _BPW)
    def _(j):
        for r in range(2):
            for v in range(8):
                sl = pl.ds(v * 16, 16)
                slab_v[j, r, sl] = slab_v[j, r, sl] * 8 + r

    abufs = (a0, a1)
    bbufs = (b0, b1)
    obufs = (o0, o1)
    sbufs = (s0, s1)
    gsems = (gs0, gs1)
    osems = (os0, os1)

    def issue_gather(j, k):
        pltpu.async_copy(t_hbm.at[slab_v.at[j, 0]], abufs[k], gsems[k])
        pltpu.async_copy(t_hbm.at[slab_v.at[j, 1]], bbufs[k], gsems[k])

    def wait_gather(k):
        pltpu.make_async_copy(t_hbm.at[pl.ds(0, _BLK)], abufs[k],
                              gsems[k]).wait()
        pltpu.make_async_copy(t_hbm.at[pl.ds(0, _BLK)], bbufs[k],
                              gsems[k]).wait()

    iota17 = cvec * 17

    def compute(k):
        a_ref, b_ref, o_ref, s_ref = abufs[k], bbufs[k], obufs[k], sbufs[k]

        @functools.partial(plsc.parallel_loop, 0, _BLK, unroll=8)
        def _(i):
            s_ref[pl.ds(i * 17, 16)] = a_ref[i] + b_ref[i]

        @functools.partial(plsc.parallel_loop, 0, 8)
        def _(g):
            gidx = iota17 + g * (16 * 17)
            gout = g * 16
            for c in range(16):
                v = plsc.load_gather(s_ref, [gidx + c])
                dst = (c >> 3) * _TILE_W + (c & 7) * _BLK
                o_ref[pl.ds(gout + dst, 16)] = v

    def issue_out(j, k):
        b = blk0 + j
        o_ref = obufs[k]
        pltpu.async_copy(o_ref.at[pl.ds(0, _TILE_W)],
                         out_hbm.at[pl.ds(b * _TILE_W, _TILE_W)], osems[k])
        pltpu.async_copy(o_ref.at[pl.ds(_TILE_W, _TILE_W)],
                         out_hbm.at[pl.ds(_HALF + b * _TILE_W, _TILE_W)],
                         osems[k])

    def wait_out(k):
        pltpu.make_async_copy(obufs[k], out_hbm.at[pl.ds(0, 2 * _TILE_W)],
                              osems[k]).wait()

    issue_gather(0, 0)
    issue_gather(1, 1)

    for j in (0, 1):
        k = j % 2
        wait_gather(k)
        compute(k)
        issue_out(j, k)
        issue_gather(j + 2, k)

    @pl.loop(2, _BPW - 2, step=2)
    def _(c):
        for k in range(2):
            j = c + k
            wait_gather(k)
            wait_out(k)
            compute(k)
            issue_out(j, k)
            issue_gather(j + 2, k)

    for j in (_BPW - 2, _BPW - 1):
        k = j % 2
        wait_gather(k)
        wait_out(k)
        compute(k)
        issue_out(j, k)

    wait_out(0)
    wait_out(1)

    @pl.when(wid < _XTRA)
    def _():
        xb = _NW * _BPW + wid
        pltpu.sync_copy(idx_hbm.at[xb], xtra_v)
        for r in range(2):
            for v in range(8):
                sl = pl.ds(v * 16, 16)
                xtra_v[r, sl] = xtra_v[r, sl] * 8 + r
        pltpu.async_copy(t_hbm.at[xtra_v.at[0]], a0, gs0)
        pltpu.async_copy(t_hbm.at[xtra_v.at[1]], b0, gs0)
        wait_gather(0)
        compute(0)
        pltpu.async_copy(o0.at[pl.ds(0, _TILE_W)],
                         out_hbm.at[pl.ds(xb * _TILE_W, _TILE_W)], os0)
        pltpu.async_copy(o0.at[pl.ds(_TILE_W, _TILE_W)],
                         out_hbm.at[pl.ds(_HALF + xb * _TILE_W, _TILE_W)],
                         os0)
        wait_out(0)


def _edge_decode(t2, idx3):
    mesh = plsc.VectorSubcoreMesh(core_axis_name="c", subcore_axis_name="s")
    f32 = jnp.float32
    run = pl.kernel(
        _edge_body,
        out_type=jax.ShapeDtypeStruct((_OUT_WORDS,), f32),
        mesh=mesh,
        compiler_params=pltpu.CompilerParams(use_tc_tiling_on_sc=False,
                                             needs_layout_passes=False),
        scratch_types=[
            pltpu.VMEM((_BPW, 2, _BLK), jnp.int32),
            pltpu.VMEM((2, _BLK), jnp.int32),
            pltpu.VMEM((_BLK, N_EDGE_FEAT), f32),
            pltpu.VMEM((_BLK, N_EDGE_FEAT), f32),
            pltpu.VMEM((_BLK, N_EDGE_FEAT), f32),
            pltpu.VMEM((_BLK, N_EDGE_FEAT), f32),
            pltpu.VMEM((2 * _TILE_W,), f32),
            pltpu.VMEM((2 * _TILE_W,), f32),
            pltpu.VMEM((_BLK * 17,), f32),
            pltpu.VMEM((_BLK * 17,), f32),
            pltpu.SemaphoreType.DMA,
            pltpu.SemaphoreType.DMA,
            pltpu.SemaphoreType.DMA,
            pltpu.SemaphoreType.DMA,
        ],
    )
    return run(t2, idx3)


def kernel(embeddings, edge_index, node_w, node_b, edge_w, edge_b):
    pad = HIDDEN - 2 * N_EDGE_FEAT
    wc = jnp.concatenate(
        [edge_w[:, :HIDDEN].T, edge_w[:, HIDDEN:].T,
         jnp.zeros((HIDDEN, pad), jnp.float32)], axis=1)
    bc = jnp.pad(edge_b, (0, HIDDEN - N_EDGE_FEAT)).reshape(1, HIDDEN)

    t = _p_table(embeddings, wc, bc)
    t2 = t.reshape(8 * N_NODES, N_EDGE_FEAT)

    idx3 = (edge_index.astype(jnp.int32)
            .reshape(2, _NBLOCKS, _BLK)
            .transpose(1, 0, 2))

    out_flat = _edge_decode(t2, idx3)
    edge_hat = (out_flat
                .reshape(2, _NBLOCKS, 8, _BLK)
                .transpose(1, 3, 0, 2)
                .reshape(N_EDGES, N_EDGE_FEAT))

    node_hat = _node_linear(embeddings, node_w.T,
                            node_b.reshape(1, N_NODE_FEAT))
    return (node_hat, edge_hat)
